# baseline (speedup 1.0000x reference)
import functools

import jax
import jax.numpy as jnp
import numpy as np
from jax import lax
from jax.experimental import pallas as pl
from jax.experimental.pallas import tpu as pltpu
from jax.experimental.pallas import tpu_sc as plsc

_BATCH = 16384
_DIM = 32
_NC = 2
_NS = 16
_NW = _NC * _NS
_BPW = _BATCH // _NW
_GCH = 128


def _sc_scores_body(uidx_hbm, iidx_hbm, utab_hbm, itab_hbm, btab_hbm,
                    out_hbm,
                    uix_v, iix_v, urow_v, irow_v, bias_v, pt_v, score_v, sem):
    wid = lax.axis_index("s") * _NC + lax.axis_index("c")
    base = wid * _BPW

    pltpu.sync_copy(uidx_hbm.at[pl.ds(base, _BPW)], uix_v)
    pltpu.sync_copy(iidx_hbm.at[pl.ds(base, _BPW)], iix_v)

    copies = []
    for j in range(_BPW // _GCH):
        sl = pl.ds(j * _GCH, _GCH)
        copies.append(pltpu.make_async_copy(
            utab_hbm.at[uix_v.at[sl]], urow_v.at[sl], sem))
        copies.append(pltpu.make_async_copy(
            itab_hbm.at[iix_v.at[sl]], irow_v.at[sl], sem))
        copies.append(pltpu.make_async_copy(
            btab_hbm.at[iix_v.at[sl]], bias_v.at[sl], sem))
    for c in copies:
        c.start()
    for c in copies:
        c.wait()

    lanes = lax.iota(jnp.int32, 16)
    tcols = lanes * _BPW

    def elem_body(b, carry):
        u0 = urow_v[b, pl.ds(0, 16)]
        u1 = urow_v[b, pl.ds(16, 16)]
        i0 = irow_v[b, pl.ds(0, 16)]
        i1 = irow_v[b, pl.ds(16, 16)]
        p = u0 * i0 + u1 * i1
        plsc.store_scatter(pt_v, [tcols + b], p)
        return carry

    lax.fori_loop(0, _BPW, elem_body, 0)

    def chunk_body(c, carry):
        row0 = c * 16
        acc = bias_v[pl.ds(row0, 16)]
        for d in range(16):
            acc = acc + pt_v[pl.ds(d * _BPW + row0, 16)]
        score_v[pl.ds(row0, 16)] = acc
        return carry

    lax.fori_loop(0, _BPW // 16, chunk_body, 0)
    pltpu.sync_copy(score_v, out_hbm.at[pl.ds(base, _BPW)])


_sc_scores = functools.partial(
    pl.kernel,
    out_type=jax.ShapeDtypeStruct((_BATCH,), jnp.float32),
    mesh=plsc.VectorSubcoreMesh(core_axis_name="c", subcore_axis_name="s"),
    compiler_params=pltpu.CompilerParams(
        needs_layout_passes=False, use_tc_tiling_on_sc=False),
    scratch_types=[
        pltpu.VMEM((_BPW,), jnp.int32),
        pltpu.VMEM((_BPW,), jnp.int32),
        pltpu.VMEM((_BPW, _DIM), jnp.float32),
        pltpu.VMEM((_BPW, _DIM), jnp.float32),
        pltpu.VMEM((_BPW,), jnp.float32),
        pltpu.VMEM((16 * _BPW,), jnp.float32),
        pltpu.VMEM((_BPW,), jnp.float32),
        pltpu.SemaphoreType.DMA,
    ],
)(_sc_scores_body)


def _loss_body(s_ref, y_ref, o_ref):
    s = s_ref[...]
    y = y_ref[...]
    per = jnp.maximum(s, 0.0) - s * y + jnp.log1p(jnp.exp(-jnp.abs(s)))
    o_ref[...] = jnp.sum(per).reshape(1, 1) / np.float32(_BATCH)


def kernel(input_user, input_item, pred_data_label,
           D_user_embeddings, D_item_embeddings, D_item_bias):
    scores = _sc_scores(input_user, input_item,
                        D_user_embeddings, D_item_embeddings, D_item_bias)
    loss = pl.pallas_call(
        _loss_body,
        out_shape=jax.ShapeDtypeStruct((1, 1), jnp.float32),
    )(scores.reshape(128, 128),
      pred_data_label.astype(jnp.float32).reshape(128, 128))
    return loss[0, 0]
